# Initial kernel scaffold; baseline (speedup 1.0000x reference)
import jax
import jax.numpy as jnp
from jax.experimental import pallas as pl


def kernel(x, EW0, Eb0, EW1, Eb1, EW2, Eb2, GW0, Gb0, GW1, Gb1, GW2, Gb2):
    raise NotImplementedError("write your pallas kernel here")


# speedup vs baseline: 1.2094x; 1.2094x over previous
import jax
import jax.numpy as jnp
from jax.experimental import pallas as pl
from jax.experimental.pallas import tpu as pltpu

E = 8
K = 2
D = 1024
H = 2048
GH = 512
B = 2048

BM = 512
NBB = B // BM


def _gate_kernel(x_ref, gw0_ref, gb0_ref, gw1_ref, gb1_ref, gw2_ref, gb2_ref,
                 tki_ref, tks_ref, wt_ref, aux_ref):
    x = x_ref[...]
    g = jnp.maximum(jnp.dot(x, gw0_ref[...], preferred_element_type=jnp.float32)
                    + gb0_ref[...], 0.0)
    g = jnp.maximum(jnp.dot(g, gw1_ref[...], preferred_element_type=jnp.float32)
                    + gb1_ref[...], 0.0)
    logits = jnp.dot(g, gw2_ref[...], preferred_element_type=jnp.float32) + gb2_ref[...]
    m = jnp.max(logits, axis=1, keepdims=True)
    unnorm = jnp.exp(logits - m)
    p = unnorm / jnp.sum(unnorm, axis=1, keepdims=True)
    iota = jax.lax.broadcasted_iota(jnp.int32, (B, E), 1)
    m0 = jnp.max(p, axis=1, keepdims=True)
    i0 = jnp.min(jnp.where(p == m0, iota, E), axis=1, keepdims=True)
    p1 = jnp.where(iota == i0, -1.0, p)
    m1 = jnp.max(p1, axis=1, keepdims=True)
    i1 = jnp.min(jnp.where(p1 == m1, iota, E), axis=1, keepdims=True)
    tki_ref[...] = jnp.concatenate([i0, i1], axis=1)
    tks_ref[...] = jnp.concatenate([m0, m1], axis=1)
    denom = m0 + m1 + 1e-9
    sel = (iota == i0) | (iota == i1)
    wt_ref[...] = jnp.where(sel, p / denom, 0.0).T
    counts = jnp.sum(sel.astype(jnp.float32), axis=0)
    load = counts * (1.0 / float(B + 1e-9))
    mload = jnp.sum(load) * (1.0 / E)
    lb = jnp.sum((load - mload) ** 2) * (1.0 / (E - 1))
    ent = -jnp.sum(p * jnp.log(p + 1e-9), axis=1)
    ent_mean = jnp.sum(ent) * (1.0 / B)
    aux = 5.0 * lb + 0.1 * ent_mean
    aux_ref[...] = jnp.broadcast_to(aux, (1, 1))


def _expert_kernel(x_ref, w0_ref, b0_ref, w1_ref, b1_ref, w2_ref, b2_ref,
                   wt_ref, out_ref):
    e = pl.program_id(1)
    xb = x_ref[...].astype(jnp.bfloat16)
    h = jnp.dot(xb, w0_ref[0], preferred_element_type=jnp.float32) + b0_ref[0]
    h = jnp.maximum(h, 0.0).astype(jnp.bfloat16)
    h = jnp.dot(h, w1_ref[0], preferred_element_type=jnp.float32) + b1_ref[0]
    h = jnp.maximum(h, 0.0).astype(jnp.bfloat16)
    y = jnp.dot(h, w2_ref[0], preferred_element_type=jnp.float32) + b2_ref[0]
    y = y * wt_ref[0, 0][:, None]

    @pl.when(e == 0)
    def _():
        out_ref[...] = y

    @pl.when(e != 0)
    def _():
        out_ref[...] += y


def kernel(x, EW0, Eb0, EW1, Eb1, EW2, Eb2, GW0, Gb0, GW1, Gb1, GW2, Gb2):
    tki, tks, wt, aux = pl.pallas_call(
        _gate_kernel,
        out_shape=[
            jax.ShapeDtypeStruct((B, K), jnp.int32),
            jax.ShapeDtypeStruct((B, K), jnp.float32),
            jax.ShapeDtypeStruct((E, B), jnp.float32),
            jax.ShapeDtypeStruct((1, 1), jnp.float32),
        ],
    )(x, GW0, Gb0.reshape(1, GH), GW1, Gb1.reshape(1, GH), GW2, Gb2.reshape(1, E))

    combined = pl.pallas_call(
        _expert_kernel,
        grid=(NBB, E),
        in_specs=[
            pl.BlockSpec((BM, D), lambda b, e: (b, 0)),
            pl.BlockSpec((1, D, H), lambda b, e: (e, 0, 0)),
            pl.BlockSpec((1, 1, H), lambda b, e: (e, 0, 0)),
            pl.BlockSpec((1, H, H), lambda b, e: (e, 0, 0)),
            pl.BlockSpec((1, 1, H), lambda b, e: (e, 0, 0)),
            pl.BlockSpec((1, H, D), lambda b, e: (e, 0, 0)),
            pl.BlockSpec((1, 1, D), lambda b, e: (e, 0, 0)),
            pl.BlockSpec((1, 1, BM), lambda b, e: (e, 0, b)),
        ],
        out_specs=pl.BlockSpec((BM, D), lambda b, e: (b, 0)),
        out_shape=jax.ShapeDtypeStruct((B, D), jnp.float32),
        compiler_params=pltpu.CompilerParams(
            dimension_semantics=("arbitrary", "arbitrary")),
    )(x, EW0.astype(jnp.bfloat16), Eb0.reshape(E, 1, H),
      EW1.astype(jnp.bfloat16), Eb1.reshape(E, 1, H),
      EW2.astype(jnp.bfloat16), Eb2.reshape(E, 1, D),
      wt.reshape(E, 1, B))

    return (combined, aux.reshape(()), tki, tks)
